# baseline (speedup 1.0000x reference)
import jax
import jax.numpy as jnp
from jax import lax
from jax.experimental import pallas as pl
from jax.experimental.pallas import tpu as pltpu
from jax.experimental.pallas import tpu_sc as plsc

_N = 10000
_E = 320000
_DIN = 128
_H = 256
_C = 64
_F = 64

_NC = 2
_NS = 16
_B = 80
_NBUF = 5
_BM = 2000
_ZR = 1000

_ROWS_E = _E // (_NC * _NS) // _B

_BP = 80
_PROWS_F = 250
_PROWS_E = 125
_RING = 5

_mesh = plsc.VectorSubcoreMesh(core_axis_name="c", subcore_axis_name="s")
_params = pltpu.CompilerParams(use_tc_tiling_on_sc=False,
                               skip_device_barrier=True)


def _m8(x):
    return pl.multiple_of(x, 8)


def _deg_body(src_e, dst_e, zeros1, out, src_v, dst_v, ones_v, acc_o, acc_i,
              sem):
    c = lax.axis_index("c")
    s = lax.axis_index("s")
    w = c * _NS + s
    pltpu.sync_copy(src_e.at[w], src_v)
    pltpu.sync_copy(dst_e.at[w], dst_v)
    for k in range(_B // 16):
        ones_v[pl.ds(k * 16, 16)] = jnp.ones((16,), jnp.float32)

    @pl.when(s < 10)
    def _():
        pltpu.sync_copy(zeros1, acc_o.at[pl.ds(_m8(s * _ZR), _ZR)])
        pltpu.sync_copy(zeros1, acc_i.at[pl.ds(_m8(s * _ZR), _ZR)])

    plsc.subcore_barrier()

    def jbody(j2, carry):
        handles = []
        for par in range(_NBUF):
            j = j2 * _NBUF + par
            handles.append(
                pltpu.async_copy(ones_v, acc_o.at[src_v.at[j]], sem, add=True))
            handles.append(
                pltpu.async_copy(ones_v, acc_i.at[dst_v.at[j]], sem, add=True))
        for h in handles:
            h.wait()
        return carry

    lax.fori_loop(0, _ROWS_E // _NBUF, jbody, 0)
    plsc.subcore_barrier()

    @pl.when(s < 10)
    def _():
        pltpu.sync_copy(acc_o.at[pl.ds(_m8(s * _ZR), _ZR)], out.at[2 * c, s])
        pltpu.sync_copy(acc_i.at[pl.ds(_m8(s * _ZR), _ZR)],
                        out.at[2 * c + 1, s])


_deg = pl.kernel(
    _deg_body,
    out_type=jax.ShapeDtypeStruct((4, 10, _ZR), jnp.float32),
    mesh=_mesh,
    compiler_params=_params,
    scratch_types=[
        pltpu.VMEM((_ROWS_E, _B), jnp.int32),
        pltpu.VMEM((_ROWS_E, _B), jnp.int32),
        pltpu.VMEM((_B,), jnp.float32),
        pltpu.VMEM_SHARED((_N,), jnp.float32),
        pltpu.VMEM_SHARED((_N,), jnp.float32),
        pltpu.SemaphoreType.DMA,
    ],
)


def _edge_loop(h_cat, src_v, dst_v, acc, bufs, gsems, nrows):
    for k in range(_RING):
        pltpu.async_copy(h_cat.at[src_v.at[k]], bufs[k], gsems[k])

    def jbody(j2, carry):
        for par in range(_RING):
            j = j2 * _RING + par
            pltpu.make_async_copy(h_cat.at[src_v.at[j]], bufs[par],
                                  gsems[par]).wait()
            pltpu.sync_copy(bufs[par], acc.at[dst_v.at[j]], add=True)

            @pl.when(j + _RING < nrows)
            def _():
                pltpu.async_copy(h_cat.at[src_v.at[j + _RING]], bufs[par],
                                 gsems[par])
        return carry

    lax.fori_loop(0, nrows // _RING, jbody, 0)


_R2 = 10
_G = 5


def _edge_loop_async(h_cat, src_v, dst_v, acc, bufs, gsems, ssems, nrows):
    for k in range(_RING):
        pltpu.async_copy(h_cat.at[src_v.at[k]], bufs[k], gsems[k])

    def jbody(j2, carry):
        for p in range(_RING):
            j = j2 * _RING + p
            pv = (p - 1) % _RING

            pltpu.make_async_copy(h_cat.at[src_v.at[j]], bufs[p],
                                  gsems[p]).wait()
            pltpu.async_copy(bufs[p], acc.at[dst_v.at[j]], ssems[p],
                             add=True)

            @pl.when(j >= 1)
            def _():
                pltpu.make_async_copy(bufs[pv], acc.at[dst_v.at[j - 1]],
                                      ssems[pv]).wait()

                @pl.when(j - 1 + _RING < nrows)
                def _():
                    pltpu.async_copy(h_cat.at[src_v.at[j - 1 + _RING]],
                                     bufs[pv], gsems[pv])
        return carry

    lax.fori_loop(0, nrows // _RING, jbody, 0)
    pltpu.make_async_copy(bufs[(nrows - 1) % _RING],
                          acc.at[dst_v.at[nrows - 1]],
                          ssems[(nrows - 1) % _RING]).wait()


def _make_prop(npass):
    def body(h_cat, idx, dst_f, zeros, out, *scr):
        c = lax.axis_index("c")
        s = lax.axis_index("s")
        src_v, dst_v = scr[0], scr[1]
        bufs = scr[2:2 + _RING]
        acc = scr[2 + _RING]
        gsems = scr[3 + _RING:3 + 2 * _RING]
        pltpu.sync_copy(dst_f.at[s], dst_v)
        for q in range(npass):
            pltpu.sync_copy(idx.at[q * 2 * _NS + c * _NS + s], src_v)

            pltpu.sync_copy(zeros, acc.at[pl.ds(s * 625, 625)])
            plsc.subcore_barrier()
            _edge_loop(h_cat, src_v, dst_v, acc, bufs, gsems, _PROWS_F)
            plsc.subcore_barrier()
            pltpu.sync_copy(acc.at[pl.ds(s * 625, 625)],
                            out.at[pl.ds((q * 2 + c) * _N + s * 625, 625)])

    return pl.kernel(
        body,
        out_type=jax.ShapeDtypeStruct((npass * 2 * _N, _F), jnp.float32),
        mesh=_mesh,
        compiler_params=_params,
        scratch_types=(
            [pltpu.VMEM((_PROWS_F, _BP), jnp.int32)] * 2
            + [pltpu.VMEM((_BP, _F), jnp.float32)] * _RING
            + [pltpu.VMEM_SHARED((_N + 8, _F), jnp.float32)]
            + [pltpu.SemaphoreType.DMA] * _RING
        ),
    )


_prop1 = _make_prop(1)
_prop2 = _make_prop(2)


def _prop3_body(h3, src_p, dst_p, zeros, out, *scr):
    c = lax.axis_index("c")
    s = lax.axis_index("s")
    src_v, dst_v = scr[0], scr[1]
    bufs = scr[2:2 + _RING]
    acc = scr[2 + _RING]
    gsems = scr[3 + _RING:3 + 2 * _RING]
    w = c * _NS + s
    pltpu.sync_copy(src_p.at[w], src_v)
    pltpu.sync_copy(dst_p.at[w], dst_v)

    pltpu.sync_copy(zeros, acc.at[pl.ds(s * 625, 625)])
    plsc.subcore_barrier()
    _edge_loop(h3, src_v, dst_v, acc, bufs, gsems, _PROWS_E)
    plsc.subcore_barrier()
    pltpu.sync_copy(acc.at[pl.ds(s * 625, 625)],
                    out.at[pl.ds(c * _N + s * 625, 625)])


_prop3 = pl.kernel(
    _prop3_body,
    out_type=jax.ShapeDtypeStruct((2 * _N, _C), jnp.float32),
    mesh=_mesh,
    compiler_params=_params,
    scratch_types=(
        [pltpu.VMEM((_PROWS_E, _BP), jnp.int32)] * 2
        + [pltpu.VMEM((_BP, _C), jnp.float32)] * _RING
        + [pltpu.VMEM_SHARED((_N + 8, _C), jnp.float32)]
        + [pltpu.SemaphoreType.DMA] * _RING
    ),
)


def _tc1_body(x_ref, dgo_ref, dgi_ref, xs_ref, ns_ref, nd_ref):
    ns = lax.rsqrt(jnp.maximum(dgo_ref[0] + dgo_ref[1], 1.0))
    nd = lax.rsqrt(jnp.maximum(dgi_ref[0] + dgi_ref[1], 1.0))
    xs = x_ref[...] * ns
    xs_ref[0] = xs[:, :_F]
    xs_ref[1] = xs[:, _F:]
    ns_ref[...] = ns
    nd_ref[...] = nd


def _tc1(x, dgo, dgi):
    return pl.pallas_call(
        _tc1_body,
        grid=(_N // _BM,),
        in_specs=[
            pl.BlockSpec((_BM, _DIN), lambda i: (i, 0)),
            pl.BlockSpec((2, _BM, 1), lambda i: (0, i, 0)),
            pl.BlockSpec((2, _BM, 1), lambda i: (0, i, 0)),
        ],
        out_specs=[
            pl.BlockSpec((2, _BM, _F), lambda i: (0, i, 0)),
            pl.BlockSpec((_BM, 1), lambda i: (i, 0)),
            pl.BlockSpec((_BM, 1), lambda i: (i, 0)),
        ],
        out_shape=[
            jax.ShapeDtypeStruct((2, _N, _F), jnp.float32),
            jax.ShapeDtypeStruct((_N, 1), jnp.float32),
            jax.ShapeDtypeStruct((_N, 1), jnp.float32),
        ],
    )(x, dgo, dgi)


def _tc2_body(a_ref, nd_ref, ns_ref, b_ref, w_ref, h_ref):
    ax = jnp.concatenate([a_ref[0], a_ref[1]], axis=1)
    h1 = jnp.dot(ax, w_ref[...], preferred_element_type=jnp.float32)
    out1 = jnp.maximum(h1 * nd_ref[...] + b_ref[...], 0.0) * ns_ref[...]
    for q in range(4):
        h_ref[q] = out1[:, q * _F:(q + 1) * _F]


def _tc2(a, nd, ns, b, w1):
    return pl.pallas_call(
        _tc2_body,
        grid=(_N // _BM,),
        in_specs=[
            pl.BlockSpec((2, _BM, _F), lambda i: (0, i, 0)),
            pl.BlockSpec((_BM, 1), lambda i: (i, 0)),
            pl.BlockSpec((_BM, 1), lambda i: (i, 0)),
            pl.BlockSpec((1, _H), lambda i: (0, 0)),
            pl.BlockSpec((_DIN, _H), lambda i: (0, 0)),
        ],
        out_specs=pl.BlockSpec((4, _BM, _F), lambda i: (0, i, 0)),
        out_shape=jax.ShapeDtypeStruct((4, _N, _F), jnp.float32),
    )(a, nd, ns, b, w1)


def _tc3_body(a_ref, nd_ref, ns_ref, b_ref, w2_ref, w3_ref, h_ref):
    agg = jnp.concatenate([a_ref[0], a_ref[1], a_ref[2], a_ref[3]], axis=1)
    h2 = jnp.dot(agg, w2_ref[...], preferred_element_type=jnp.float32)
    out2 = jnp.maximum(h2 * nd_ref[...] + b_ref[...], 0.0) * ns_ref[...]
    h_ref[...] = jnp.dot(out2, w3_ref[...], preferred_element_type=jnp.float32)


def _tc3(a, nd, ns, b, w2, w3):
    return pl.pallas_call(
        _tc3_body,
        grid=(_N // _BM,),
        in_specs=[
            pl.BlockSpec((4, _BM, _F), lambda i: (0, i, 0)),
            pl.BlockSpec((_BM, 1), lambda i: (i, 0)),
            pl.BlockSpec((_BM, 1), lambda i: (i, 0)),
            pl.BlockSpec((1, _H), lambda i: (0, 0)),
            pl.BlockSpec((_H, _H), lambda i: (0, 0)),
            pl.BlockSpec((_H, _C), lambda i: (0, 0)),
        ],
        out_specs=pl.BlockSpec((_BM, _C), lambda i: (i, 0)),
        out_shape=jax.ShapeDtypeStruct((_N, _C), jnp.float32),
    )(a, nd, ns, b, w2, w3)


def _tcf_body(p_ref, nd_ref, b_ref, o_ref):
    o_ref[...] = (p_ref[0] + p_ref[1]) * nd_ref[...] + b_ref[...]


def _tcf(p, nd, b):
    return pl.pallas_call(
        _tcf_body,
        grid=(_N // _BM,),
        in_specs=[
            pl.BlockSpec((2, _BM, _C), lambda i: (0, i, 0)),
            pl.BlockSpec((_BM, 1), lambda i: (i, 0)),
            pl.BlockSpec((1, _C), lambda i: (0, 0)),
        ],
        out_specs=pl.BlockSpec((_BM, _C), lambda i: (i, 0)),
        out_shape=jax.ShapeDtypeStruct((_N, _C), jnp.float32),
    )(p, nd, b)


@jax.jit
def _run(x, edge_index, W1, b1, W2, b2, W3, b3):
    src = edge_index[0]
    dst = edge_index[1]
    src_e = src.reshape(_NC * _NS, _ROWS_E, _B)
    dst_e = dst.reshape(_NC * _NS, _ROWS_E, _B)

    padf = _PROWS_F * _BP - _E // _NS
    pade = _PROWS_E * _BP - _E // (_NC * _NS)
    rotf = jnp.arange(padf, dtype=jnp.int32) % 8
    rote = jnp.arange(pade, dtype=jnp.int32) % 8
    src_f = jnp.concatenate(
        [src.reshape(_NS, -1), jnp.broadcast_to(rotf, (_NS, padf))],
        axis=1).reshape(_NS, _PROWS_F, _BP)
    dst_f = jnp.concatenate(
        [dst.reshape(_NS, -1), jnp.broadcast_to(_N + rotf, (_NS, padf))],
        axis=1).reshape(_NS, _PROWS_F, _BP)
    idx4 = jnp.concatenate(
        [src_f, src_f + _N, src_f + 2 * _N, src_f + 3 * _N], axis=0)
    nw = _NC * _NS
    src_p = jnp.concatenate(
        [src.reshape(nw, -1), jnp.broadcast_to(rote, (nw, pade))],
        axis=1).reshape(nw, _PROWS_E, _BP)
    dst_p = jnp.concatenate(
        [dst.reshape(nw, -1), jnp.broadcast_to(_N + rote, (nw, pade))],
        axis=1).reshape(nw, _PROWS_E, _BP)

    zeros1 = jnp.zeros((_ZR,), jnp.float32)
    zeros64 = jnp.zeros((625, _F), jnp.float32)

    d = _deg(src_e, dst_e, zeros1).reshape(2, 2, _N)
    dgo = d[:, 0, :][..., None]
    dgi = d[:, 1, :][..., None]

    xs, ns, nd = _tc1(x, dgo, dgi)
    ax = _prop1(xs.reshape(2 * _N, _F), idx4[:2 * _NS], dst_f, zeros64)
    h1 = _tc2(ax.reshape(2, _N, _F), nd, ns, b1.reshape(1, _H), W1)
    a2 = _prop2(h1.reshape(4 * _N, _F), idx4, dst_f, zeros64)
    h3 = _tc3(a2.reshape(4, _N, _F), nd, ns, b2.reshape(1, _H), W2, W3)
    p = _prop3(h3, src_p, dst_p, zeros64)
    return _tcf(p.reshape(2, _N, _C), nd, b3.reshape(1, _C))


def kernel(x, edge_index, W1, b1, W2, b2, W3, b3):
    return _run(x, edge_index, W1, b1, W2, b2, W3, b3)


# speedup vs baseline: 1.0209x; 1.0209x over previous
import jax
import jax.numpy as jnp
from jax import lax
from jax.experimental import pallas as pl
from jax.experimental.pallas import tpu as pltpu
from jax.experimental.pallas import tpu_sc as plsc

_N = 10000
_E = 320000
_DIN = 128
_H = 256
_C = 64
_F = 64

_NC = 2
_NS = 16
_B = 80
_NBUF = 5
_BM = 2000
_ZR = 1000

_ROWS_E = _E // (_NC * _NS) // _B

_BP = 80
_PROWS_F = 250
_PROWS_E = 125
_RING = 5

_mesh = plsc.VectorSubcoreMesh(core_axis_name="c", subcore_axis_name="s")
_params = pltpu.CompilerParams(use_tc_tiling_on_sc=False,
                               skip_device_barrier=True)


def _m8(x):
    return pl.multiple_of(x, 8)


def _deg_body(src_e, dst_e, zeros1, out, src_v, dst_v, ones_v, acc_o, acc_i,
              sem):
    c = lax.axis_index("c")
    s = lax.axis_index("s")
    w = c * _NS + s
    pltpu.sync_copy(src_e.at[w], src_v)
    pltpu.sync_copy(dst_e.at[w], dst_v)
    for k in range(_B // 16):
        ones_v[pl.ds(k * 16, 16)] = jnp.ones((16,), jnp.float32)

    @pl.when(s < 10)
    def _():
        pltpu.sync_copy(zeros1, acc_o.at[pl.ds(_m8(s * _ZR), _ZR)])
        pltpu.sync_copy(zeros1, acc_i.at[pl.ds(_m8(s * _ZR), _ZR)])

    plsc.subcore_barrier()

    def jbody(j2, carry):
        handles = []
        for par in range(_NBUF):
            j = j2 * _NBUF + par
            handles.append(
                pltpu.async_copy(ones_v, acc_o.at[src_v.at[j]], sem, add=True))
            handles.append(
                pltpu.async_copy(ones_v, acc_i.at[dst_v.at[j]], sem, add=True))
        for h in handles:
            h.wait()
        return carry

    lax.fori_loop(0, _ROWS_E // _NBUF, jbody, 0)
    plsc.subcore_barrier()

    @pl.when(s < 10)
    def _():
        pltpu.sync_copy(acc_o.at[pl.ds(_m8(s * _ZR), _ZR)], out.at[2 * c, s])
        pltpu.sync_copy(acc_i.at[pl.ds(_m8(s * _ZR), _ZR)],
                        out.at[2 * c + 1, s])


_deg = pl.kernel(
    _deg_body,
    out_type=jax.ShapeDtypeStruct((4, 10, _ZR), jnp.float32),
    mesh=_mesh,
    compiler_params=_params,
    scratch_types=[
        pltpu.VMEM((_ROWS_E, _B), jnp.int32),
        pltpu.VMEM((_ROWS_E, _B), jnp.int32),
        pltpu.VMEM((_B,), jnp.float32),
        pltpu.VMEM_SHARED((_N,), jnp.float32),
        pltpu.VMEM_SHARED((_N,), jnp.float32),
        pltpu.SemaphoreType.DMA,
    ],
)


def _edge_loop(h_cat, src_v, dst_v, acc, bufs, gsems, nrows):
    for k in range(_RING):
        pltpu.async_copy(h_cat.at[src_v.at[k]], bufs[k], gsems[k])

    def jbody(j2, carry):
        for par in range(_RING):
            j = j2 * _RING + par
            pltpu.make_async_copy(h_cat.at[src_v.at[j]], bufs[par],
                                  gsems[par]).wait()
            pltpu.sync_copy(bufs[par], acc.at[dst_v.at[j]], add=True)

            @pl.when(j + _RING < nrows)
            def _():
                pltpu.async_copy(h_cat.at[src_v.at[j + _RING]], bufs[par],
                                 gsems[par])
        return carry

    lax.fori_loop(0, nrows // _RING, jbody, 0)


_R2 = 10
_G = 5


def _edge_loop_async(h_cat, src_v, dst_v, acc, bufs, gsems, ssems, nrows):
    for k in range(_RING):
        pltpu.async_copy(h_cat.at[src_v.at[k]], bufs[k], gsems[k])

    def jbody(j2, carry):
        for p in range(_RING):
            j = j2 * _RING + p
            pv = (p - 1) % _RING

            pltpu.make_async_copy(h_cat.at[src_v.at[j]], bufs[p],
                                  gsems[p]).wait()
            pltpu.async_copy(bufs[p], acc.at[dst_v.at[j]], ssems[p],
                             add=True)

            @pl.when(j >= 1)
            def _():
                pltpu.make_async_copy(bufs[pv], acc.at[dst_v.at[j - 1]],
                                      ssems[pv]).wait()

                @pl.when(j - 1 + _RING < nrows)
                def _():
                    pltpu.async_copy(h_cat.at[src_v.at[j - 1 + _RING]],
                                     bufs[pv], gsems[pv])
        return carry

    lax.fori_loop(0, nrows // _RING, jbody, 0)
    pltpu.make_async_copy(bufs[(nrows - 1) % _RING],
                          acc.at[dst_v.at[nrows - 1]],
                          ssems[(nrows - 1) % _RING]).wait()


def _make_prop(npass):
    def body(h_cat, idx, dst_f, zeros, out, *scr):
        c = lax.axis_index("c")
        s = lax.axis_index("s")
        src_v, dst_v = scr[0], scr[1]
        bufs = scr[2:2 + _RING]
        acc = scr[2 + _RING]
        gsems = scr[3 + _RING:3 + 2 * _RING]
        pltpu.sync_copy(dst_f.at[s], dst_v)
        for q in range(npass):
            pltpu.sync_copy(idx.at[q * 2 * _NS + c * _NS + s], src_v)

            @pl.when(s < 10)
            def _():
                pltpu.sync_copy(zeros, acc.at[pl.ds(_m8(s * _ZR), _ZR)])

            plsc.subcore_barrier()
            _edge_loop(h_cat, src_v, dst_v, acc, bufs, gsems, _PROWS_F)
            plsc.subcore_barrier()

            @pl.when(s < 10)
            def _():
                pltpu.sync_copy(
                    acc.at[pl.ds(_m8(s * _ZR), _ZR)],
                    out.at[pl.ds(_m8((q * 2 + c) * _N + s * _ZR), _ZR)])

    return pl.kernel(
        body,
        out_type=jax.ShapeDtypeStruct((npass * 2 * _N, _F), jnp.float32),
        mesh=_mesh,
        compiler_params=_params,
        scratch_types=(
            [pltpu.VMEM((_PROWS_F, _BP), jnp.int32)] * 2
            + [pltpu.VMEM((_BP, _F), jnp.float32)] * _RING
            + [pltpu.VMEM_SHARED((_N + 8, _F), jnp.float32)]
            + [pltpu.SemaphoreType.DMA] * _RING
        ),
    )


_prop1 = _make_prop(1)
_prop2 = _make_prop(2)


def _prop3_body(h3, src_p, dst_p, zeros, out, *scr):
    c = lax.axis_index("c")
    s = lax.axis_index("s")
    src_v, dst_v = scr[0], scr[1]
    bufs = scr[2:2 + _RING]
    acc = scr[2 + _RING]
    gsems = scr[3 + _RING:3 + 2 * _RING]
    w = c * _NS + s
    pltpu.sync_copy(src_p.at[w], src_v)
    pltpu.sync_copy(dst_p.at[w], dst_v)

    @pl.when(s < 10)
    def _():
        pltpu.sync_copy(zeros, acc.at[pl.ds(_m8(s * _ZR), _ZR)])

    plsc.subcore_barrier()
    _edge_loop(h3, src_v, dst_v, acc, bufs, gsems, _PROWS_E)
    plsc.subcore_barrier()

    @pl.when(s < 10)
    def _():
        pltpu.sync_copy(acc.at[pl.ds(_m8(s * _ZR), _ZR)],
                        out.at[pl.ds(_m8(c * _N + s * _ZR), _ZR)])


_prop3 = pl.kernel(
    _prop3_body,
    out_type=jax.ShapeDtypeStruct((2 * _N, _C), jnp.float32),
    mesh=_mesh,
    compiler_params=_params,
    scratch_types=(
        [pltpu.VMEM((_PROWS_E, _BP), jnp.int32)] * 2
        + [pltpu.VMEM((_BP, _C), jnp.float32)] * _RING
        + [pltpu.VMEM_SHARED((_N + 8, _C), jnp.float32)]
        + [pltpu.SemaphoreType.DMA] * _RING
    ),
)


def _tc1_body(x_ref, dgo_ref, dgi_ref, xs_ref, ns_ref, nd_ref):
    ns = lax.rsqrt(jnp.maximum(dgo_ref[0] + dgo_ref[1], 1.0))
    nd = lax.rsqrt(jnp.maximum(dgi_ref[0] + dgi_ref[1], 1.0))
    xs = x_ref[...] * ns
    xs_ref[0] = xs[:, :_F]
    xs_ref[1] = xs[:, _F:]
    ns_ref[...] = ns
    nd_ref[...] = nd


def _tc1(x, dgo, dgi):
    return pl.pallas_call(
        _tc1_body,
        grid=(_N // _BM,),
        in_specs=[
            pl.BlockSpec((_BM, _DIN), lambda i: (i, 0)),
            pl.BlockSpec((2, _BM, 1), lambda i: (0, i, 0)),
            pl.BlockSpec((2, _BM, 1), lambda i: (0, i, 0)),
        ],
        out_specs=[
            pl.BlockSpec((2, _BM, _F), lambda i: (0, i, 0)),
            pl.BlockSpec((_BM, 1), lambda i: (i, 0)),
            pl.BlockSpec((_BM, 1), lambda i: (i, 0)),
        ],
        out_shape=[
            jax.ShapeDtypeStruct((2, _N, _F), jnp.float32),
            jax.ShapeDtypeStruct((_N, 1), jnp.float32),
            jax.ShapeDtypeStruct((_N, 1), jnp.float32),
        ],
    )(x, dgo, dgi)


def _tc2_body(a_ref, nd_ref, ns_ref, b_ref, w_ref, h_ref):
    ax = jnp.concatenate([a_ref[0], a_ref[1]], axis=1)
    h1 = jnp.dot(ax, w_ref[...], preferred_element_type=jnp.float32)
    out1 = jnp.maximum(h1 * nd_ref[...] + b_ref[...], 0.0) * ns_ref[...]
    for q in range(4):
        h_ref[q] = out1[:, q * _F:(q + 1) * _F]


def _tc2(a, nd, ns, b, w1):
    return pl.pallas_call(
        _tc2_body,
        grid=(_N // _BM,),
        in_specs=[
            pl.BlockSpec((2, _BM, _F), lambda i: (0, i, 0)),
            pl.BlockSpec((_BM, 1), lambda i: (i, 0)),
            pl.BlockSpec((_BM, 1), lambda i: (i, 0)),
            pl.BlockSpec((1, _H), lambda i: (0, 0)),
            pl.BlockSpec((_DIN, _H), lambda i: (0, 0)),
        ],
        out_specs=pl.BlockSpec((4, _BM, _F), lambda i: (0, i, 0)),
        out_shape=jax.ShapeDtypeStruct((4, _N, _F), jnp.float32),
    )(a, nd, ns, b, w1)


def _tc3_body(a_ref, nd_ref, ns_ref, b_ref, w2_ref, w3_ref, h_ref):
    agg = jnp.concatenate([a_ref[0], a_ref[1], a_ref[2], a_ref[3]], axis=1)
    h2 = jnp.dot(agg, w2_ref[...], preferred_element_type=jnp.float32)
    out2 = jnp.maximum(h2 * nd_ref[...] + b_ref[...], 0.0) * ns_ref[...]
    h_ref[...] = jnp.dot(out2, w3_ref[...], preferred_element_type=jnp.float32)


def _tc3(a, nd, ns, b, w2, w3):
    return pl.pallas_call(
        _tc3_body,
        grid=(_N // _BM,),
        in_specs=[
            pl.BlockSpec((4, _BM, _F), lambda i: (0, i, 0)),
            pl.BlockSpec((_BM, 1), lambda i: (i, 0)),
            pl.BlockSpec((_BM, 1), lambda i: (i, 0)),
            pl.BlockSpec((1, _H), lambda i: (0, 0)),
            pl.BlockSpec((_H, _H), lambda i: (0, 0)),
            pl.BlockSpec((_H, _C), lambda i: (0, 0)),
        ],
        out_specs=pl.BlockSpec((_BM, _C), lambda i: (i, 0)),
        out_shape=jax.ShapeDtypeStruct((_N, _C), jnp.float32),
    )(a, nd, ns, b, w2, w3)


def _tcf_body(p_ref, nd_ref, b_ref, o_ref):
    o_ref[...] = (p_ref[0] + p_ref[1]) * nd_ref[...] + b_ref[...]


def _tcf(p, nd, b):
    return pl.pallas_call(
        _tcf_body,
        grid=(_N // _BM,),
        in_specs=[
            pl.BlockSpec((2, _BM, _C), lambda i: (0, i, 0)),
            pl.BlockSpec((_BM, 1), lambda i: (i, 0)),
            pl.BlockSpec((1, _C), lambda i: (0, 0)),
        ],
        out_specs=pl.BlockSpec((_BM, _C), lambda i: (i, 0)),
        out_shape=jax.ShapeDtypeStruct((_N, _C), jnp.float32),
    )(p, nd, b)


@jax.jit
def _run(x, edge_index, W1, b1, W2, b2, W3, b3):
    src = edge_index[0]
    dst = edge_index[1]
    src_e = src.reshape(_NC * _NS, _ROWS_E, _B)
    dst_e = dst.reshape(_NC * _NS, _ROWS_E, _B)

    padf = _PROWS_F * _BP - _E // _NS
    pade = _PROWS_E * _BP - _E // (_NC * _NS)
    rotf = jnp.arange(padf, dtype=jnp.int32) % 8
    rote = jnp.arange(pade, dtype=jnp.int32) % 8
    src_f = jnp.concatenate(
        [src.reshape(_NS, -1), jnp.broadcast_to(rotf, (_NS, padf))],
        axis=1).reshape(_NS, _PROWS_F, _BP)
    dst_f = jnp.concatenate(
        [dst.reshape(_NS, -1), jnp.broadcast_to(_N + rotf, (_NS, padf))],
        axis=1).reshape(_NS, _PROWS_F, _BP)
    idx4 = jnp.concatenate(
        [src_f, src_f + _N, src_f + 2 * _N, src_f + 3 * _N], axis=0)
    nw = _NC * _NS
    src_p = jnp.concatenate(
        [src.reshape(nw, -1), jnp.broadcast_to(rote, (nw, pade))],
        axis=1).reshape(nw, _PROWS_E, _BP)
    dst_p = jnp.concatenate(
        [dst.reshape(nw, -1), jnp.broadcast_to(_N + rote, (nw, pade))],
        axis=1).reshape(nw, _PROWS_E, _BP)

    zeros1 = jnp.zeros((_ZR,), jnp.float32)
    zeros64 = jnp.zeros((_ZR, _F), jnp.float32)

    d = _deg(src_e, dst_e, zeros1).reshape(2, 2, _N)
    dgo = d[:, 0, :][..., None]
    dgi = d[:, 1, :][..., None]

    xs, ns, nd = _tc1(x, dgo, dgi)
    ax = _prop1(xs.reshape(2 * _N, _F), idx4[:2 * _NS], dst_f, zeros64)
    h1 = _tc2(ax.reshape(2, _N, _F), nd, ns, b1.reshape(1, _H), W1)
    a2 = _prop2(h1.reshape(4 * _N, _F), idx4, dst_f, zeros64)
    h3 = _tc3(a2.reshape(4, _N, _F), nd, ns, b2.reshape(1, _H), W2, W3)
    p = _prop3(h3, src_p, dst_p, zeros64)
    return _tcf(p.reshape(2, _N, _C), nd, b3.reshape(1, _C))


def kernel(x, edge_index, W1, b1, W2, b2, W3, b3):
    return _run(x, edge_index, W1, b1, W2, b2, W3, b3)


# speedup vs baseline: 1.0217x; 1.0008x over previous
import jax
import jax.numpy as jnp
from jax import lax
from jax.experimental import pallas as pl
from jax.experimental.pallas import tpu as pltpu
from jax.experimental.pallas import tpu_sc as plsc

_N = 10000
_E = 320000
_DIN = 128
_H = 256
_C = 64
_F = 64

_NC = 2
_NS = 16
_B = 80
_NBUF = 5
_BM = 2000
_ZR = 1000

_ROWS_E = _E // (_NC * _NS) // _B

_BP = 80
_PROWS_F = 250
_PROWS_E = 125
_RING = 5

_mesh = plsc.VectorSubcoreMesh(core_axis_name="c", subcore_axis_name="s")
_params = pltpu.CompilerParams(use_tc_tiling_on_sc=False)


def _m8(x):
    return pl.multiple_of(x, 8)


def _deg_body(src_e, dst_e, zeros1, out, src_v, dst_v, ones_v, acc_o, acc_i,
              sem):
    c = lax.axis_index("c")
    s = lax.axis_index("s")
    w = c * _NS + s
    pltpu.sync_copy(src_e.at[w], src_v)
    pltpu.sync_copy(dst_e.at[w], dst_v)
    for k in range(_B // 16):
        ones_v[pl.ds(k * 16, 16)] = jnp.ones((16,), jnp.float32)

    @pl.when(s < 10)
    def _():
        pltpu.sync_copy(zeros1, acc_o.at[pl.ds(_m8(s * _ZR), _ZR)])
        pltpu.sync_copy(zeros1, acc_i.at[pl.ds(_m8(s * _ZR), _ZR)])

    plsc.subcore_barrier()

    def jbody(j2, carry):
        handles = []
        for par in range(_NBUF):
            j = j2 * _NBUF + par
            handles.append(
                pltpu.async_copy(ones_v, acc_o.at[src_v.at[j]], sem, add=True))
            handles.append(
                pltpu.async_copy(ones_v, acc_i.at[dst_v.at[j]], sem, add=True))
        for h in handles:
            h.wait()
        return carry

    lax.fori_loop(0, _ROWS_E // _NBUF, jbody, 0)
    plsc.subcore_barrier()

    @pl.when(s < 10)
    def _():
        pltpu.sync_copy(acc_o.at[pl.ds(_m8(s * _ZR), _ZR)], out.at[2 * c, s])
        pltpu.sync_copy(acc_i.at[pl.ds(_m8(s * _ZR), _ZR)],
                        out.at[2 * c + 1, s])


_deg = pl.kernel(
    _deg_body,
    out_type=jax.ShapeDtypeStruct((4, 10, _ZR), jnp.float32),
    mesh=_mesh,
    compiler_params=_params,
    scratch_types=[
        pltpu.VMEM((_ROWS_E, _B), jnp.int32),
        pltpu.VMEM((_ROWS_E, _B), jnp.int32),
        pltpu.VMEM((_B,), jnp.float32),
        pltpu.VMEM_SHARED((_N,), jnp.float32),
        pltpu.VMEM_SHARED((_N,), jnp.float32),
        pltpu.SemaphoreType.DMA,
    ],
)


def _edge_loop(h_cat, src_v, dst_v, acc, bufs, gsems, nrows):
    for k in range(_RING):
        pltpu.async_copy(h_cat.at[src_v.at[k]], bufs[k], gsems[k])

    def jbody(j2, carry):
        for par in range(_RING):
            j = j2 * _RING + par
            pltpu.make_async_copy(h_cat.at[src_v.at[j]], bufs[par],
                                  gsems[par]).wait()
            pltpu.sync_copy(bufs[par], acc.at[dst_v.at[j]], add=True)

            @pl.when(j + _RING < nrows)
            def _():
                pltpu.async_copy(h_cat.at[src_v.at[j + _RING]], bufs[par],
                                 gsems[par])
        return carry

    lax.fori_loop(0, nrows // _RING, jbody, 0)


def _make_prop(npass):
    def body(h_cat, idx, dst_f, zeros, out, *scr):
        c = lax.axis_index("c")
        s = lax.axis_index("s")
        src_v, dst_v = scr[0], scr[1]
        bufs = scr[2:2 + _RING]
        acc = scr[2 + _RING]
        gsems = scr[3 + _RING:3 + 2 * _RING]
        pltpu.sync_copy(dst_f.at[s], dst_v)
        for q in range(npass):
            pltpu.sync_copy(idx.at[q * 2 * _NS + c * _NS + s], src_v)

            @pl.when(s < 10)
            def _():
                pltpu.sync_copy(zeros, acc.at[pl.ds(_m8(s * _ZR), _ZR)])

            plsc.subcore_barrier()
            _edge_loop(h_cat, src_v, dst_v, acc, bufs, gsems, _PROWS_F)
            plsc.subcore_barrier()

            @pl.when(s < 10)
            def _():
                pltpu.sync_copy(
                    acc.at[pl.ds(_m8(s * _ZR), _ZR)],
                    out.at[pl.ds(_m8((q * 2 + c) * _N + s * _ZR), _ZR)])

    return pl.kernel(
        body,
        out_type=jax.ShapeDtypeStruct((npass * 2 * _N, _F), jnp.float32),
        mesh=_mesh,
        compiler_params=_params,
        scratch_types=(
            [pltpu.VMEM((_PROWS_F, _BP), jnp.int32)] * 2
            + [pltpu.VMEM((_BP, _F), jnp.float32)] * _RING
            + [pltpu.VMEM_SHARED((_N + 8, _F), jnp.float32)]
            + [pltpu.SemaphoreType.DMA] * _RING
        ),
    )


_prop1 = _make_prop(1)
_prop2 = _make_prop(2)


def _prop3_body(h3, src_p, dst_p, zeros, out, *scr):
    c = lax.axis_index("c")
    s = lax.axis_index("s")
    src_v, dst_v = scr[0], scr[1]
    bufs = scr[2:2 + _RING]
    acc = scr[2 + _RING]
    gsems = scr[3 + _RING:3 + 2 * _RING]
    w = c * _NS + s
    pltpu.sync_copy(src_p.at[w], src_v)
    pltpu.sync_copy(dst_p.at[w], dst_v)

    @pl.when(s < 10)
    def _():
        pltpu.sync_copy(zeros, acc.at[pl.ds(_m8(s * _ZR), _ZR)])

    plsc.subcore_barrier()
    _edge_loop(h3, src_v, dst_v, acc, bufs, gsems, _PROWS_E)
    plsc.subcore_barrier()

    @pl.when(s < 10)
    def _():
        pltpu.sync_copy(acc.at[pl.ds(_m8(s * _ZR), _ZR)],
                        out.at[pl.ds(_m8(c * _N + s * _ZR), _ZR)])


_prop3 = pl.kernel(
    _prop3_body,
    out_type=jax.ShapeDtypeStruct((2 * _N, _C), jnp.float32),
    mesh=_mesh,
    compiler_params=_params,
    scratch_types=(
        [pltpu.VMEM((_PROWS_E, _BP), jnp.int32)] * 2
        + [pltpu.VMEM((_BP, _C), jnp.float32)] * _RING
        + [pltpu.VMEM_SHARED((_N + 8, _C), jnp.float32)]
        + [pltpu.SemaphoreType.DMA] * _RING
    ),
)


def _tc1_body(x_ref, dgo_ref, dgi_ref, xs_ref, ns_ref, nd_ref):
    ns = lax.rsqrt(jnp.maximum(dgo_ref[0] + dgo_ref[1], 1.0))
    nd = lax.rsqrt(jnp.maximum(dgi_ref[0] + dgi_ref[1], 1.0))
    xs = x_ref[...] * ns
    xs_ref[0] = xs[:, :_F]
    xs_ref[1] = xs[:, _F:]
    ns_ref[...] = ns
    nd_ref[...] = nd


def _tc1(x, dgo, dgi):
    return pl.pallas_call(
        _tc1_body,
        grid=(_N // _BM,),
        in_specs=[
            pl.BlockSpec((_BM, _DIN), lambda i: (i, 0)),
            pl.BlockSpec((2, _BM, 1), lambda i: (0, i, 0)),
            pl.BlockSpec((2, _BM, 1), lambda i: (0, i, 0)),
        ],
        out_specs=[
            pl.BlockSpec((2, _BM, _F), lambda i: (0, i, 0)),
            pl.BlockSpec((_BM, 1), lambda i: (i, 0)),
            pl.BlockSpec((_BM, 1), lambda i: (i, 0)),
        ],
        out_shape=[
            jax.ShapeDtypeStruct((2, _N, _F), jnp.float32),
            jax.ShapeDtypeStruct((_N, 1), jnp.float32),
            jax.ShapeDtypeStruct((_N, 1), jnp.float32),
        ],
    )(x, dgo, dgi)


def _tc2_body(a_ref, nd_ref, ns_ref, b_ref, w_ref, h_ref):
    ax = jnp.concatenate([a_ref[0], a_ref[1]], axis=1)
    h1 = jnp.dot(ax, w_ref[...], preferred_element_type=jnp.float32)
    out1 = jnp.maximum(h1 * nd_ref[...] + b_ref[...], 0.0) * ns_ref[...]
    for q in range(4):
        h_ref[q] = out1[:, q * _F:(q + 1) * _F]


def _tc2(a, nd, ns, b, w1):
    return pl.pallas_call(
        _tc2_body,
        grid=(_N // _BM,),
        in_specs=[
            pl.BlockSpec((2, _BM, _F), lambda i: (0, i, 0)),
            pl.BlockSpec((_BM, 1), lambda i: (i, 0)),
            pl.BlockSpec((_BM, 1), lambda i: (i, 0)),
            pl.BlockSpec((1, _H), lambda i: (0, 0)),
            pl.BlockSpec((_DIN, _H), lambda i: (0, 0)),
        ],
        out_specs=pl.BlockSpec((4, _BM, _F), lambda i: (0, i, 0)),
        out_shape=jax.ShapeDtypeStruct((4, _N, _F), jnp.float32),
    )(a, nd, ns, b, w1)


def _tc3_body(a_ref, nd_ref, ns_ref, b_ref, w2_ref, w3_ref, h_ref):
    agg = jnp.concatenate([a_ref[0], a_ref[1], a_ref[2], a_ref[3]], axis=1)
    h2 = jnp.dot(agg, w2_ref[...], preferred_element_type=jnp.float32)
    out2 = jnp.maximum(h2 * nd_ref[...] + b_ref[...], 0.0) * ns_ref[...]
    h_ref[...] = jnp.dot(out2, w3_ref[...], preferred_element_type=jnp.float32)


def _tc3(a, nd, ns, b, w2, w3):
    return pl.pallas_call(
        _tc3_body,
        grid=(_N // _BM,),
        in_specs=[
            pl.BlockSpec((4, _BM, _F), lambda i: (0, i, 0)),
            pl.BlockSpec((_BM, 1), lambda i: (i, 0)),
            pl.BlockSpec((_BM, 1), lambda i: (i, 0)),
            pl.BlockSpec((1, _H), lambda i: (0, 0)),
            pl.BlockSpec((_H, _H), lambda i: (0, 0)),
            pl.BlockSpec((_H, _C), lambda i: (0, 0)),
        ],
        out_specs=pl.BlockSpec((_BM, _C), lambda i: (i, 0)),
        out_shape=jax.ShapeDtypeStruct((_N, _C), jnp.float32),
    )(a, nd, ns, b, w2, w3)


def _tcf_body(p_ref, nd_ref, b_ref, o_ref):
    o_ref[...] = (p_ref[0] + p_ref[1]) * nd_ref[...] + b_ref[...]


def _tcf(p, nd, b):
    return pl.pallas_call(
        _tcf_body,
        grid=(_N // _BM,),
        in_specs=[
            pl.BlockSpec((2, _BM, _C), lambda i: (0, i, 0)),
            pl.BlockSpec((_BM, 1), lambda i: (i, 0)),
            pl.BlockSpec((1, _C), lambda i: (0, 0)),
        ],
        out_specs=pl.BlockSpec((_BM, _C), lambda i: (i, 0)),
        out_shape=jax.ShapeDtypeStruct((_N, _C), jnp.float32),
    )(p, nd, b)


@jax.jit
def _run(x, edge_index, W1, b1, W2, b2, W3, b3):
    src = edge_index[0]
    dst = edge_index[1]
    src_e = src.reshape(_NC * _NS, _ROWS_E, _B)
    dst_e = dst.reshape(_NC * _NS, _ROWS_E, _B)

    padf = _PROWS_F * _BP - _E // _NS
    pade = _PROWS_E * _BP - _E // (_NC * _NS)
    rotf = jnp.arange(padf, dtype=jnp.int32) % 8
    rote = jnp.arange(pade, dtype=jnp.int32) % 8
    src_f = jnp.concatenate(
        [src.reshape(_NS, -1), jnp.broadcast_to(rotf, (_NS, padf))],
        axis=1).reshape(_NS, _PROWS_F, _BP)
    dst_f = jnp.concatenate(
        [dst.reshape(_NS, -1), jnp.broadcast_to(_N + rotf, (_NS, padf))],
        axis=1).reshape(_NS, _PROWS_F, _BP)
    idx4 = jnp.concatenate(
        [src_f, src_f + _N, src_f + 2 * _N, src_f + 3 * _N], axis=0)
    nw = _NC * _NS
    src_p = jnp.concatenate(
        [src.reshape(nw, -1), jnp.broadcast_to(rote, (nw, pade))],
        axis=1).reshape(nw, _PROWS_E, _BP)
    dst_p = jnp.concatenate(
        [dst.reshape(nw, -1), jnp.broadcast_to(_N + rote, (nw, pade))],
        axis=1).reshape(nw, _PROWS_E, _BP)

    zeros1 = jnp.zeros((_ZR,), jnp.float32)
    zeros64 = jnp.zeros((_ZR, _F), jnp.float32)

    d = _deg(src_e, dst_e, zeros1).reshape(2, 2, _N)
    dgo = d[:, 0, :][..., None]
    dgi = d[:, 1, :][..., None]

    xs, ns, nd = _tc1(x, dgo, dgi)
    ax = _prop1(xs.reshape(2 * _N, _F), idx4[:2 * _NS], dst_f, zeros64)
    h1 = _tc2(ax.reshape(2, _N, _F), nd, ns, b1.reshape(1, _H), W1)
    a2 = _prop2(h1.reshape(4 * _N, _F), idx4, dst_f, zeros64)
    h3 = _tc3(a2.reshape(4, _N, _F), nd, ns, b2.reshape(1, _H), W2, W3)
    p = _prop3(h3, src_p, dst_p, zeros64)
    return _tcf(p.reshape(2, _N, _C), nd, b3.reshape(1, _C))


def kernel(x, edge_index, W1, b1, W2, b2, W3, b3):
    return _run(x, edge_index, W1, b1, W2, b2, W3, b3)
